# baseline (device time: 200087 ns/iter reference)
import jax
import jax.numpy as jnp
from jax import lax
from jax.experimental import pallas as pl
from jax.experimental.pallas import tpu as pltpu

Z = 4


def kernel(x, pi):
    def body(x_ref, pi_ref, out_ref, send_sem, recv_sem):
        my_x = lax.axis_index("x")
        my_y = lax.axis_index("y")
        my_z = lax.axis_index("z")
        r = (Z - pi_ref[0]) % Z
        dst_z = (my_z - r) % Z
        rdma = pltpu.make_async_remote_copy(
            src_ref=x_ref,
            dst_ref=out_ref,
            send_sem=send_sem,
            recv_sem=recv_sem,
            device_id=(my_x, my_y, dst_z),
            device_id_type=pl.DeviceIdType.MESH,
        )
        rdma.start()
        rdma.wait()

    return pl.pallas_call(
        body,
        out_shape=jax.ShapeDtypeStruct(x.shape, x.dtype),
        in_specs=[
            pl.BlockSpec(memory_space=pltpu.VMEM),
            pl.BlockSpec(memory_space=pltpu.SMEM),
        ],
        out_specs=pl.BlockSpec(memory_space=pltpu.VMEM),
        scratch_shapes=[
            pltpu.SemaphoreType.DMA,
            pltpu.SemaphoreType.DMA,
        ],
    )(x, pi)


# device time: 114803 ns/iter; 1.7429x vs baseline; 1.7429x over previous
import jax
import jax.numpy as jnp
from jax import lax
from jax.experimental import pallas as pl
from jax.experimental.pallas import tpu as pltpu

Z = 4
M = 2048
Q = M // 4


def kernel(x, pi):
    def body(
        x_ref,
        pi_ref,
        out_ref,
        p1_send,
        p1_recv,
        p2_send,
        p2_recv,
        p3a_send,
        p3a_recv,
        p3b_send,
        p3b_recv,
    ):
        my_x = lax.axis_index("x")
        my_y = lax.axis_index("y")
        my_z = lax.axis_index("z")
        r = (Z - pi_ref[0]) % Z
        dst_z = (my_z - r) % Z

        ypar = my_y % 2
        q = 2 * my_x + ypar
        qq = 2 * my_x + (1 - ypar)
        y_partner = my_y + 1 - 2 * ypar
        x_partner = 1 - my_x

        p1 = pltpu.make_async_remote_copy(
            src_ref=x_ref.at[:, pl.ds(q * Q, Q), :],
            dst_ref=out_ref.at[:, pl.ds(q * Q, Q), :],
            send_sem=p1_send,
            recv_sem=p1_recv,
            device_id=(my_x, my_y, dst_z),
            device_id_type=pl.DeviceIdType.MESH,
        )
        p1.start()
        p1.wait()

        p2 = pltpu.make_async_remote_copy(
            src_ref=out_ref.at[:, pl.ds(q * Q, Q), :],
            dst_ref=out_ref.at[:, pl.ds(q * Q, Q), :],
            send_sem=p2_send,
            recv_sem=p2_recv,
            device_id=(my_x, y_partner, my_z),
            device_id_type=pl.DeviceIdType.MESH,
        )
        p2.start()
        p3a = pltpu.make_async_remote_copy(
            src_ref=out_ref.at[:, pl.ds(q * Q, Q), :],
            dst_ref=out_ref.at[:, pl.ds(q * Q, Q), :],
            send_sem=p3a_send,
            recv_sem=p3a_recv,
            device_id=(x_partner, my_y, my_z),
            device_id_type=pl.DeviceIdType.MESH,
        )
        p3a.start()
        p2.wait()

        p3b = pltpu.make_async_remote_copy(
            src_ref=out_ref.at[:, pl.ds(qq * Q, Q), :],
            dst_ref=out_ref.at[:, pl.ds(qq * Q, Q), :],
            send_sem=p3b_send,
            recv_sem=p3b_recv,
            device_id=(x_partner, my_y, my_z),
            device_id_type=pl.DeviceIdType.MESH,
        )
        p3b.start()
        p3a.wait()
        p3b.wait()

    return pl.pallas_call(
        body,
        out_shape=jax.ShapeDtypeStruct(x.shape, x.dtype),
        in_specs=[
            pl.BlockSpec(memory_space=pltpu.VMEM),
            pl.BlockSpec(memory_space=pltpu.SMEM),
        ],
        out_specs=pl.BlockSpec(memory_space=pltpu.VMEM),
        scratch_shapes=[pltpu.SemaphoreType.DMA] * 8,
    )(x, pi)


# device time: 96300 ns/iter; 2.0777x vs baseline; 1.1921x over previous
import jax
import jax.numpy as jnp
from jax import lax
from jax.experimental import pallas as pl
from jax.experimental.pallas import tpu as pltpu

Z = 4
M = 2048
Q = M // 4
C = 4
QC = Q // C


def kernel(x, pi):
    def body(
        x_ref,
        pi_ref,
        out_ref,
        p1_send,
        p1_recv,
        p2_send,
        p2_recv,
        p3a_send,
        p3a_recv,
        p3b_send,
        p3b_recv,
    ):
        my_x = lax.axis_index("x")
        my_y = lax.axis_index("y")
        my_z = lax.axis_index("z")
        r = (Z - pi_ref[0]) % Z
        dst_z = (my_z - r) % Z

        ypar = my_y % 2
        q = 2 * my_x + ypar
        qq = 2 * my_x + (1 - ypar)
        y_partner = my_y + 1 - 2 * ypar
        x_partner = 1 - my_x

        def copy(rows_start, nbr, send_sems, recv_sems, c, src=None):
            src = x_ref if src is None else src
            return pltpu.make_async_remote_copy(
                src_ref=src.at[:, pl.ds(rows_start, QC), :],
                dst_ref=out_ref.at[:, pl.ds(rows_start, QC), :],
                send_sem=send_sems.at[c],
                recv_sem=recv_sems.at[c],
                device_id=nbr,
                device_id_type=pl.DeviceIdType.MESH,
            )

        col = (my_x, my_y, dst_z)
        ynbr = (my_x, y_partner, my_z)
        xnbr = (x_partner, my_y, my_z)

        p1 = [copy(q * Q + c * QC, col, p1_send, p1_recv, c) for c in range(C)]
        for d in p1:
            d.start()

        p2, p3a = [], []
        for c in range(C):
            p1[c].wait()
            d2 = copy(q * Q + c * QC, ynbr, p2_send, p2_recv, c, src=out_ref)
            d2.start()
            p2.append(d2)
            d3 = copy(q * Q + c * QC, xnbr, p3a_send, p3a_recv, c, src=out_ref)
            d3.start()
            p3a.append(d3)

        p3b = []
        for c in range(C):
            p2[c].wait()
            d4 = copy(qq * Q + c * QC, xnbr, p3b_send, p3b_recv, c, src=out_ref)
            d4.start()
            p3b.append(d4)

        for d in p3a:
            d.wait()
        for d in p3b:
            d.wait()

    return pl.pallas_call(
        body,
        out_shape=jax.ShapeDtypeStruct(x.shape, x.dtype),
        in_specs=[
            pl.BlockSpec(memory_space=pltpu.VMEM),
            pl.BlockSpec(memory_space=pltpu.SMEM),
        ],
        out_specs=pl.BlockSpec(memory_space=pltpu.VMEM),
        scratch_shapes=[pltpu.SemaphoreType.DMA((C,))] * 8,
    )(x, pi)
